# baseline (speedup 1.0000x reference)
import jax
import jax.numpy as jnp
from jax import lax
from jax.experimental import pallas as pl
from jax.experimental.pallas import tpu as pltpu
from jax.experimental.pallas import tpu_sc as plsc

USER_NUM = 1000000
ITEM_NUM = 1000000
EMB_DIM = 16
BATCH = 16384

NUM_CORES = 2
NUM_SUBCORES = 16
NUM_WORKERS = NUM_CORES * NUM_SUBCORES
ROWS_PER_WORKER = BATCH // NUM_WORKERS

L = 16
PHYS_W = 128
ROWS_PER_PHYS = PHYS_W // EMB_DIM
CHUNK = 256
N_CHUNK = ROWS_PER_WORKER // CHUNK


def _sc_logits_kernel(ue2_hbm, ie2_hbm, ibias_hbm, uidx_hbm, iidx_hbm,
                      logits_hbm,
                      uhi_v, ihi_v, ulo_v, ilo_v, iraw_v, tmp_v,
                      bias_v, logits_v, ug_v, ig_v,
                      sem_u, sem_i, sem_b):
    wid = lax.axis_index("s") * NUM_CORES + lax.axis_index("c")
    base = wid * ROWS_PER_WORKER

    pltpu.sync_copy(uidx_hbm.at[pl.ds(base, ROWS_PER_WORKER)], tmp_v)
    pltpu.sync_copy(iidx_hbm.at[pl.ds(base, ROWS_PER_WORKER)], iraw_v)

    cb = pltpu.async_copy(ibias_hbm.at[iraw_v], bias_v, sem_b)

    @pl.loop(0, ROWS_PER_WORKER, step=L)
    def _(j):
        uv = tmp_v[pl.ds(j, L)]
        iv = iraw_v[pl.ds(j, L)]
        uhi_v[pl.ds(j, L)] = jnp.right_shift(uv, 3)
        ulo_v[pl.ds(j, L)] = jnp.bitwise_and(uv, 7) * EMB_DIM
        ihi_v[pl.ds(j, L)] = jnp.right_shift(iv, 3)
        ilo_v[pl.ds(j, L)] = jnp.bitwise_and(iv, 7) * EMB_DIM

    cb.wait()

    for c in range(N_CHUNK):
        cu = pltpu.async_copy(
            ue2_hbm.at[uhi_v.at[pl.ds(c * CHUNK, CHUNK)]], ug_v, sem_u)
        ci = pltpu.async_copy(
            ie2_hbm.at[ihi_v.at[pl.ds(c * CHUNK, CHUNK)]], ig_v, sem_i)
        cu.wait()
        ci.wait()

        @pl.loop(0, CHUNK, step=L)
        def _(g):
            jvec = jnp.arange(L, dtype=jnp.int32) + g
            uo = ulo_v[pl.ds(c * CHUNK + g, L)]
            io = ilo_v[pl.ds(c * CHUNK + g, L)]
            acc = bias_v[pl.ds(c * CHUNK + g, L)]
            for d in range(EMB_DIM):
                ut = plsc.load_gather(ug_v, [jvec, uo + d])
                it = plsc.load_gather(ig_v, [jvec, io + d])
                acc = acc + ut * it
            logits_v[pl.ds(c * CHUNK + g, L)] = acc

    pltpu.sync_copy(logits_v, logits_hbm.at[pl.ds(base, ROWS_PER_WORKER)])


def _sc_logits(ue2, ie2, item_bias, user, item):
    mesh = plsc.VectorSubcoreMesh(core_axis_name="c", subcore_axis_name="s")
    k = pl.kernel(
        _sc_logits_kernel,
        out_type=jax.ShapeDtypeStruct((BATCH,), jnp.float32),
        mesh=mesh,
        compiler_params=pltpu.CompilerParams(needs_layout_passes=False),
        scratch_types=[
            pltpu.VMEM((ROWS_PER_WORKER,), jnp.int32),
            pltpu.VMEM((ROWS_PER_WORKER,), jnp.int32),
            pltpu.VMEM((ROWS_PER_WORKER,), jnp.int32),
            pltpu.VMEM((ROWS_PER_WORKER,), jnp.int32),
            pltpu.VMEM((ROWS_PER_WORKER,), jnp.int32),
            pltpu.VMEM((ROWS_PER_WORKER,), jnp.int32),
            pltpu.VMEM((ROWS_PER_WORKER,), jnp.float32),
            pltpu.VMEM((ROWS_PER_WORKER,), jnp.float32),
            pltpu.VMEM((CHUNK, PHYS_W), jnp.float32),
            pltpu.VMEM((CHUNK, PHYS_W), jnp.float32),
            pltpu.SemaphoreType.DMA,
            pltpu.SemaphoreType.DMA,
            pltpu.SemaphoreType.DMA,
        ],
    )
    return k(ue2, ie2, item_bias, user, item)


def _tc_bce_kernel(x_ref, t_ref, o_ref):
    logits = x_ref[...]
    t = t_ref[...]
    per = (jnp.maximum(logits, 0.0) - logits * t
           + jnp.log1p(jnp.exp(-jnp.abs(logits))))
    o_ref[...] = jnp.reshape(jnp.sum(per) * (1.0 / BATCH), (1, 1))


@jax.jit
def kernel(user, item, label, user_embeddings, item_embeddings, item_bias):
    user = user.astype(jnp.int32)
    item = item.astype(jnp.int32)
    one = jnp.where(label[0] < 2.0, jnp.float32(1.0), jnp.float32(2.0))
    ue2 = (user_embeddings * one).reshape(USER_NUM * EMB_DIM // PHYS_W, PHYS_W)
    ie2 = (item_embeddings * one).reshape(ITEM_NUM * EMB_DIM // PHYS_W, PHYS_W)
    logits = _sc_logits(ue2, ie2, item_bias, user, item)
    loss = pl.pallas_call(
        _tc_bce_kernel,
        out_shape=jax.ShapeDtypeStruct((1, 1), jnp.float32),
    )(logits.reshape(128, 128), label.reshape(128, 128))
    return loss.reshape(())


# speedup vs baseline: 1.5234x; 1.5234x over previous
import jax
import jax.numpy as jnp
from jax import lax
from jax.experimental import pallas as pl
from jax.experimental.pallas import tpu as pltpu
from jax.experimental.pallas import tpu_sc as plsc

USER_NUM = 1000000
ITEM_NUM = 1000000
EMB_DIM = 16
BATCH = 16384

NUM_CORES = 2
NUM_SUBCORES = 16
NUM_WORKERS = NUM_CORES * NUM_SUBCORES
ROWS_PER_WORKER = BATCH // NUM_WORKERS

L = 16
PHYS_W = 128
ROWS_PER_PHYS = PHYS_W // EMB_DIM
CHUNK = 256
N_CHUNK = ROWS_PER_WORKER // CHUNK


def _sc_logits_kernel(ue2_hbm, ie2_hbm, ibias_hbm, uidx_hbm, iidx_hbm,
                      logits_hbm,
                      uhi_v, ihi_v, ulo_v, ilo_v, iraw_v, tmp_v,
                      bias_v, logits_v, ug_v, ig_v,
                      sem_u, sem_i, sem_b):
    wid = lax.axis_index("s") * NUM_CORES + lax.axis_index("c")
    base = wid * ROWS_PER_WORKER

    pltpu.sync_copy(uidx_hbm.at[pl.ds(base, ROWS_PER_WORKER)], tmp_v)
    pltpu.sync_copy(iidx_hbm.at[pl.ds(base, ROWS_PER_WORKER)], iraw_v)

    cb = pltpu.async_copy(ibias_hbm.at[iraw_v], bias_v, sem_b)

    @pl.loop(0, ROWS_PER_WORKER, step=L)
    def _(j):
        uv = tmp_v[pl.ds(j, L)]
        iv = iraw_v[pl.ds(j, L)]
        uhi_v[pl.ds(j, L)] = jnp.right_shift(uv, 3)
        ulo_v[pl.ds(j, L)] = jnp.bitwise_and(uv, 7) * EMB_DIM
        ihi_v[pl.ds(j, L)] = jnp.right_shift(iv, 3)
        ilo_v[pl.ds(j, L)] = jnp.bitwise_and(iv, 7) * EMB_DIM

    cb.wait()

    for c in range(N_CHUNK):
        cu = pltpu.async_copy(
            ue2_hbm.at[uhi_v.at[pl.ds(c * CHUNK, CHUNK)]], ug_v, sem_u)
        ci = pltpu.async_copy(
            ie2_hbm.at[ihi_v.at[pl.ds(c * CHUNK, CHUNK)]], ig_v, sem_i)
        cu.wait()
        ci.wait()

        @pl.loop(0, CHUNK, step=L)
        def _(g):
            jvec = jnp.arange(L, dtype=jnp.int32) + g
            uo = ulo_v[pl.ds(c * CHUNK + g, L)]
            io = ilo_v[pl.ds(c * CHUNK + g, L)]
            acc = bias_v[pl.ds(c * CHUNK + g, L)]
            for d in range(EMB_DIM):
                ut = plsc.load_gather(ug_v, [jvec, uo + d])
                it = plsc.load_gather(ig_v, [jvec, io + d])
                acc = acc + ut * it
            logits_v[pl.ds(c * CHUNK + g, L)] = acc

    pltpu.sync_copy(logits_v, logits_hbm.at[pl.ds(base, ROWS_PER_WORKER)])


def _sc_logits(ue2, ie2, item_bias, user, item):
    mesh = plsc.VectorSubcoreMesh(core_axis_name="c", subcore_axis_name="s")
    k = pl.kernel(
        _sc_logits_kernel,
        out_type=jax.ShapeDtypeStruct((BATCH,), jnp.float32),
        mesh=mesh,
        compiler_params=pltpu.CompilerParams(needs_layout_passes=False),
        scratch_types=[
            pltpu.VMEM((ROWS_PER_WORKER,), jnp.int32),
            pltpu.VMEM((ROWS_PER_WORKER,), jnp.int32),
            pltpu.VMEM((ROWS_PER_WORKER,), jnp.int32),
            pltpu.VMEM((ROWS_PER_WORKER,), jnp.int32),
            pltpu.VMEM((ROWS_PER_WORKER,), jnp.int32),
            pltpu.VMEM((ROWS_PER_WORKER,), jnp.int32),
            pltpu.VMEM((ROWS_PER_WORKER,), jnp.float32),
            pltpu.VMEM((ROWS_PER_WORKER,), jnp.float32),
            pltpu.VMEM((CHUNK, PHYS_W), jnp.float32),
            pltpu.VMEM((CHUNK, PHYS_W), jnp.float32),
            pltpu.SemaphoreType.DMA,
            pltpu.SemaphoreType.DMA,
            pltpu.SemaphoreType.DMA,
        ],
    )
    return k(ue2, ie2, item_bias, user, item)


def _tc_bce_kernel(x_ref, t_ref, o_ref):
    logits = x_ref[...]
    t = t_ref[...]
    per = (jnp.maximum(logits, 0.0) - logits * t
           + jnp.log1p(jnp.exp(-jnp.abs(logits))))
    o_ref[...] = jnp.reshape(jnp.sum(per) * (1.0 / BATCH), (1, 1))


@jax.jit
def kernel(user, item, label, user_embeddings, item_embeddings, item_bias):
    user = user.astype(jnp.int32)
    item = item.astype(jnp.int32)
    ue2 = user_embeddings.reshape(USER_NUM // ROWS_PER_PHYS, PHYS_W)
    ie2 = item_embeddings.reshape(ITEM_NUM // ROWS_PER_PHYS, PHYS_W)
    logits = _sc_logits(ue2, ie2, item_bias, user, item)
    loss = pl.pallas_call(
        _tc_bce_kernel,
        out_shape=jax.ShapeDtypeStruct((1, 1), jnp.float32),
    )(logits.reshape(128, 128), label.reshape(128, 128))
    return loss.reshape(())


# speedup vs baseline: 2.2422x; 1.4718x over previous
import jax
import jax.numpy as jnp
from jax import lax
from jax.experimental import pallas as pl
from jax.experimental.pallas import tpu as pltpu
from jax.experimental.pallas import tpu_sc as plsc

USER_NUM = 1000000
ITEM_NUM = 1000000
EMB_DIM = 16
BATCH = 16384

NUM_CORES = 2
NUM_SUBCORES = 16
NUM_WORKERS = NUM_CORES * NUM_SUBCORES
ROWS_PER_WORKER = BATCH // NUM_WORKERS

L = 16
PHYS_W = 128
CHUNK = 256
N_CHUNK = ROWS_PER_WORKER // CHUNK

RELAYOUT_W = 8192
N_BLK = -(-USER_NUM // RELAYOUT_W)
PACK_ROWS = N_BLK * (RELAYOUT_W // 8)


def _sc_logits_kernel(ue2_hbm, ie2_hbm, ibias_hbm, uidx_hbm, iidx_hbm,
                      logits_hbm,
                      uhi_v, ihi_v, ulo_v, ilo_v, iraw_v, tmp_v,
                      bias_v, logits_v, ug_v, ig_v,
                      sem_u, sem_i, sem_b):
    wid = lax.axis_index("s") * NUM_CORES + lax.axis_index("c")
    base = wid * ROWS_PER_WORKER

    pltpu.sync_copy(uidx_hbm.at[pl.ds(base, ROWS_PER_WORKER)], tmp_v)
    pltpu.sync_copy(iidx_hbm.at[pl.ds(base, ROWS_PER_WORKER)], iraw_v)

    cb = pltpu.async_copy(ibias_hbm.at[iraw_v], bias_v, sem_b)

    @pl.loop(0, ROWS_PER_WORKER, step=L)
    def _(j):
        uv = tmp_v[pl.ds(j, L)]
        iv = iraw_v[pl.ds(j, L)]
        uhi_v[pl.ds(j, L)] = (jnp.right_shift(uv, 13) * 1024
                              + jnp.bitwise_and(uv, 1023))
        ulo_v[pl.ds(j, L)] = jnp.bitwise_and(
            jnp.right_shift(uv, 10), 7) * EMB_DIM
        ihi_v[pl.ds(j, L)] = (jnp.right_shift(iv, 13) * 1024
                              + jnp.bitwise_and(iv, 1023))
        ilo_v[pl.ds(j, L)] = jnp.bitwise_and(
            jnp.right_shift(iv, 10), 7) * EMB_DIM

    cb.wait()

    for c in range(N_CHUNK):
        cu = pltpu.async_copy(
            ue2_hbm.at[uhi_v.at[pl.ds(c * CHUNK, CHUNK)]], ug_v, sem_u)
        ci = pltpu.async_copy(
            ie2_hbm.at[ihi_v.at[pl.ds(c * CHUNK, CHUNK)]], ig_v, sem_i)
        cu.wait()
        ci.wait()

        @pl.loop(0, CHUNK, step=L)
        def _(g):
            jvec = jnp.arange(L, dtype=jnp.int32) + g
            uo = ulo_v[pl.ds(c * CHUNK + g, L)]
            io = ilo_v[pl.ds(c * CHUNK + g, L)]
            acc = bias_v[pl.ds(c * CHUNK + g, L)]
            for d in range(EMB_DIM):
                ut = plsc.load_gather(ug_v, [jvec, uo + d])
                it = plsc.load_gather(ig_v, [jvec, io + d])
                acc = acc + ut * it
            logits_v[pl.ds(c * CHUNK + g, L)] = acc

    pltpu.sync_copy(logits_v, logits_hbm.at[pl.ds(base, ROWS_PER_WORKER)])


def _sc_logits(ue2, ie2, item_bias, user, item):
    mesh = plsc.VectorSubcoreMesh(core_axis_name="c", subcore_axis_name="s")
    k = pl.kernel(
        _sc_logits_kernel,
        out_type=jax.ShapeDtypeStruct((BATCH,), jnp.float32),
        mesh=mesh,
        compiler_params=pltpu.CompilerParams(needs_layout_passes=False),
        scratch_types=[
            pltpu.VMEM((ROWS_PER_WORKER,), jnp.int32),
            pltpu.VMEM((ROWS_PER_WORKER,), jnp.int32),
            pltpu.VMEM((ROWS_PER_WORKER,), jnp.int32),
            pltpu.VMEM((ROWS_PER_WORKER,), jnp.int32),
            pltpu.VMEM((ROWS_PER_WORKER,), jnp.int32),
            pltpu.VMEM((ROWS_PER_WORKER,), jnp.int32),
            pltpu.VMEM((ROWS_PER_WORKER,), jnp.float32),
            pltpu.VMEM((ROWS_PER_WORKER,), jnp.float32),
            pltpu.VMEM((CHUNK, PHYS_W), jnp.float32),
            pltpu.VMEM((CHUNK, PHYS_W), jnp.float32),
            pltpu.SemaphoreType.DMA,
            pltpu.SemaphoreType.DMA,
            pltpu.SemaphoreType.DMA,
        ],
    )
    return k(ue2, ie2, item_bias, user, item)


def _tc_relayout_kernel(xu_ref, xi_ref, ou_ref, oi_ref):
    eye = jnp.eye(EMB_DIM, dtype=jnp.float32)
    for x_ref, o_ref in ((xu_ref, ou_ref), (xi_ref, oi_ref)):
        x = x_ref[...]
        panels = []
        for k in range(8):
            xk = x[:, k * 1024:(k + 1) * 1024]
            panels.append(lax.dot_general(
                xk, eye, (((0,), (0,)), ((), ())),
                preferred_element_type=jnp.float32))
        o_ref[...] = jnp.concatenate(panels, axis=1)


def _tc_relayout(ue_t, ie_t):
    return pl.pallas_call(
        _tc_relayout_kernel,
        grid=(N_BLK,),
        in_specs=[pl.BlockSpec((EMB_DIM, RELAYOUT_W), lambda c: (0, c)),
                  pl.BlockSpec((EMB_DIM, RELAYOUT_W), lambda c: (0, c))],
        out_specs=[pl.BlockSpec((RELAYOUT_W // 8, PHYS_W), lambda c: (c, 0)),
                   pl.BlockSpec((RELAYOUT_W // 8, PHYS_W), lambda c: (c, 0))],
        out_shape=[jax.ShapeDtypeStruct((PACK_ROWS, PHYS_W), jnp.float32),
                   jax.ShapeDtypeStruct((PACK_ROWS, PHYS_W), jnp.float32)],
    )(ue_t, ie_t)


def _tc_bce_kernel(x_ref, t_ref, o_ref):
    logits = x_ref[...]
    t = t_ref[...]
    per = (jnp.maximum(logits, 0.0) - logits * t
           + jnp.log1p(jnp.exp(-jnp.abs(logits))))
    o_ref[...] = jnp.reshape(jnp.sum(per) * (1.0 / BATCH), (1, 1))


@jax.jit
def kernel(user, item, label, user_embeddings, item_embeddings, item_bias):
    user = user.astype(jnp.int32)
    item = item.astype(jnp.int32)
    ue2, ie2 = _tc_relayout(user_embeddings.T, item_embeddings.T)
    logits = _sc_logits(ue2, ie2, item_bias, user, item)
    loss = pl.pallas_call(
        _tc_bce_kernel,
        out_shape=jax.ShapeDtypeStruct((1, 1), jnp.float32),
    )(logits.reshape(128, 128), label.reshape(128, 128))
    return loss.reshape(())


# speedup vs baseline: 10.6755x; 4.7613x over previous
import jax
import jax.numpy as jnp
from jax import lax
from jax.experimental import pallas as pl
from jax.experimental.pallas import tpu as pltpu
from jax.experimental.pallas import tpu_sc as plsc

USER_NUM = 1000000
ITEM_NUM = 1000000
EMB_DIM = 16
BATCH = 16384

NUM_CORES = 2
NUM_SUBCORES = 16
NUM_WORKERS = NUM_CORES * NUM_SUBCORES
ROWS_PER_WORKER = BATCH // NUM_WORKERS

L = 16
PHYS_W = 128
CHUNK = 256
N_CHUNK = ROWS_PER_WORKER // CHUNK

RELAYOUT_W = 32768
PANEL = RELAYOUT_W // 8
N_BLK = -(-USER_NUM // RELAYOUT_W)
PACK_ROWS = N_BLK * PANEL


def _sc_logits_kernel(ue2_hbm, ie2_hbm, ibias_hbm, uidx_hbm, iidx_hbm,
                      logits_hbm,
                      uhi_v, ihi_v, ulo_v, ilo_v, iraw_v, tmp_v,
                      bias_v, logits_v, ug_v, ig_v,
                      sem_u, sem_i, sem_b):
    wid = lax.axis_index("s") * NUM_CORES + lax.axis_index("c")
    base = wid * ROWS_PER_WORKER

    pltpu.sync_copy(uidx_hbm.at[pl.ds(base, ROWS_PER_WORKER)], tmp_v)
    pltpu.sync_copy(iidx_hbm.at[pl.ds(base, ROWS_PER_WORKER)], iraw_v)

    cb = pltpu.async_copy(ibias_hbm.at[iraw_v], bias_v, sem_b)

    @pl.loop(0, ROWS_PER_WORKER, step=L)
    def _(j):
        uv = tmp_v[pl.ds(j, L)]
        iv = iraw_v[pl.ds(j, L)]
        uhi_v[pl.ds(j, L)] = (jnp.right_shift(uv, 15) * PANEL
                              + jnp.bitwise_and(uv, PANEL - 1))
        ulo_v[pl.ds(j, L)] = jnp.bitwise_and(
            jnp.right_shift(uv, 12), 7) * EMB_DIM
        ihi_v[pl.ds(j, L)] = (jnp.right_shift(iv, 15) * PANEL
                              + jnp.bitwise_and(iv, PANEL - 1))
        ilo_v[pl.ds(j, L)] = jnp.bitwise_and(
            jnp.right_shift(iv, 12), 7) * EMB_DIM

    cb.wait()

    for c in range(N_CHUNK):
        cu = pltpu.async_copy(
            ue2_hbm.at[uhi_v.at[pl.ds(c * CHUNK, CHUNK)]], ug_v, sem_u)
        ci = pltpu.async_copy(
            ie2_hbm.at[ihi_v.at[pl.ds(c * CHUNK, CHUNK)]], ig_v, sem_i)
        cu.wait()
        ci.wait()

        @pl.loop(0, CHUNK, step=L)
        def _(g):
            jvec = jnp.arange(L, dtype=jnp.int32) + g
            uo = ulo_v[pl.ds(c * CHUNK + g, L)]
            io = ilo_v[pl.ds(c * CHUNK + g, L)]
            acc = bias_v[pl.ds(c * CHUNK + g, L)]
            for d in range(EMB_DIM):
                ut = plsc.load_gather(ug_v, [jvec, uo + d])
                it = plsc.load_gather(ig_v, [jvec, io + d])
                acc = acc + ut * it
            logits_v[pl.ds(c * CHUNK + g, L)] = acc

    pltpu.sync_copy(logits_v, logits_hbm.at[pl.ds(base, ROWS_PER_WORKER)])


def _sc_logits(ue2, ie2, item_bias, user, item):
    mesh = plsc.VectorSubcoreMesh(core_axis_name="c", subcore_axis_name="s")
    k = pl.kernel(
        _sc_logits_kernel,
        out_type=jax.ShapeDtypeStruct((BATCH,), jnp.float32),
        mesh=mesh,
        compiler_params=pltpu.CompilerParams(needs_layout_passes=False),
        scratch_types=[
            pltpu.VMEM((ROWS_PER_WORKER,), jnp.int32),
            pltpu.VMEM((ROWS_PER_WORKER,), jnp.int32),
            pltpu.VMEM((ROWS_PER_WORKER,), jnp.int32),
            pltpu.VMEM((ROWS_PER_WORKER,), jnp.int32),
            pltpu.VMEM((ROWS_PER_WORKER,), jnp.int32),
            pltpu.VMEM((ROWS_PER_WORKER,), jnp.int32),
            pltpu.VMEM((ROWS_PER_WORKER,), jnp.float32),
            pltpu.VMEM((ROWS_PER_WORKER,), jnp.float32),
            pltpu.VMEM((CHUNK, PHYS_W), jnp.float32),
            pltpu.VMEM((CHUNK, PHYS_W), jnp.float32),
            pltpu.SemaphoreType.DMA,
            pltpu.SemaphoreType.DMA,
            pltpu.SemaphoreType.DMA,
        ],
    )
    return k(ue2, ie2, item_bias, user, item)


def _tc_relayout_kernel(xu_ref, xi_ref, ou_ref, oi_ref):
    for x_ref, o_ref in ((xu_ref, ou_ref), (xi_ref, oi_ref)):
        x = x_ref[...]
        y = jnp.concatenate(
            [x[:, k * PANEL:(k + 1) * PANEL] for k in range(8)],
            axis=0)
        o_ref[...] = y.T


def _tc_relayout(ue_t, ie_t):
    return pl.pallas_call(
        _tc_relayout_kernel,
        grid=(N_BLK,),
        in_specs=[pl.BlockSpec((EMB_DIM, RELAYOUT_W), lambda c: (0, c)),
                  pl.BlockSpec((EMB_DIM, RELAYOUT_W), lambda c: (0, c))],
        out_specs=[pl.BlockSpec((PANEL, PHYS_W), lambda c: (c, 0)),
                   pl.BlockSpec((PANEL, PHYS_W), lambda c: (c, 0))],
        out_shape=[jax.ShapeDtypeStruct((PACK_ROWS, PHYS_W), jnp.float32),
                   jax.ShapeDtypeStruct((PACK_ROWS, PHYS_W), jnp.float32)],
    )(ue_t, ie_t)


def _tc_bce_kernel(x_ref, t_ref, o_ref):
    logits = x_ref[...]
    t = t_ref[...]
    per = (jnp.maximum(logits, 0.0) - logits * t
           + jnp.log1p(jnp.exp(-jnp.abs(logits))))
    o_ref[...] = jnp.reshape(jnp.sum(per) * (1.0 / BATCH), (1, 1))


@jax.jit
def kernel(user, item, label, user_embeddings, item_embeddings, item_bias):
    user = user.astype(jnp.int32)
    item = item.astype(jnp.int32)
    ue2, ie2 = _tc_relayout(user_embeddings.T, item_embeddings.T)
    logits = _sc_logits(ue2, ie2, item_bias, user, item)
    loss = pl.pallas_call(
        _tc_bce_kernel,
        out_shape=jax.ShapeDtypeStruct((1, 1), jnp.float32),
    )(logits.reshape(128, 128), label.reshape(128, 128))
    return loss.reshape(())


# speedup vs baseline: 10.8611x; 1.0174x over previous
import jax
import jax.numpy as jnp
from jax import lax
from jax.experimental import pallas as pl
from jax.experimental.pallas import tpu as pltpu
from jax.experimental.pallas import tpu_sc as plsc

USER_NUM = 1000000
ITEM_NUM = 1000000
EMB_DIM = 16
BATCH = 16384

NUM_CORES = 2
NUM_SUBCORES = 16
NUM_WORKERS = NUM_CORES * NUM_SUBCORES
ROWS_PER_WORKER = BATCH // NUM_WORKERS

L = 16
PHYS_W = 128
CHUNK = 256
N_CHUNK = ROWS_PER_WORKER // CHUNK

RELAYOUT_W = 65536
PANEL = RELAYOUT_W // 8
N_BLK = -(-USER_NUM // RELAYOUT_W)
PACK_ROWS = N_BLK * PANEL
W_SHIFT = RELAYOUT_W.bit_length() - 1
P_SHIFT = PANEL.bit_length() - 1


def _sc_logits_kernel(ue2_hbm, ie2_hbm, ibias_hbm, uidx_hbm, iidx_hbm,
                      logits_hbm,
                      uhi_v, ihi_v, ulo_v, ilo_v, iraw_v, tmp_v,
                      bias_v, logits_v, ug_v, ig_v,
                      sem_u, sem_i, sem_b):
    wid = lax.axis_index("s") * NUM_CORES + lax.axis_index("c")
    base = wid * ROWS_PER_WORKER

    pltpu.sync_copy(uidx_hbm.at[pl.ds(base, ROWS_PER_WORKER)], tmp_v)
    pltpu.sync_copy(iidx_hbm.at[pl.ds(base, ROWS_PER_WORKER)], iraw_v)

    cb = pltpu.async_copy(ibias_hbm.at[iraw_v], bias_v, sem_b)

    @pl.loop(0, ROWS_PER_WORKER, step=L)
    def _(j):
        uv = tmp_v[pl.ds(j, L)]
        iv = iraw_v[pl.ds(j, L)]
        uhi_v[pl.ds(j, L)] = (jnp.right_shift(uv, W_SHIFT) * PANEL
                              + jnp.bitwise_and(uv, PANEL - 1))
        ulo_v[pl.ds(j, L)] = jnp.bitwise_and(
            jnp.right_shift(uv, P_SHIFT), 7) * EMB_DIM
        ihi_v[pl.ds(j, L)] = (jnp.right_shift(iv, W_SHIFT) * PANEL
                              + jnp.bitwise_and(iv, PANEL - 1))
        ilo_v[pl.ds(j, L)] = jnp.bitwise_and(
            jnp.right_shift(iv, P_SHIFT), 7) * EMB_DIM

    cb.wait()

    for c in range(N_CHUNK):
        cu = pltpu.async_copy(
            ue2_hbm.at[uhi_v.at[pl.ds(c * CHUNK, CHUNK)]], ug_v, sem_u)
        ci = pltpu.async_copy(
            ie2_hbm.at[ihi_v.at[pl.ds(c * CHUNK, CHUNK)]], ig_v, sem_i)
        cu.wait()
        ci.wait()

        @pl.loop(0, CHUNK, step=L)
        def _(g):
            jvec = jnp.arange(L, dtype=jnp.int32) + g
            uo = ulo_v[pl.ds(c * CHUNK + g, L)]
            io = ilo_v[pl.ds(c * CHUNK + g, L)]
            acc = bias_v[pl.ds(c * CHUNK + g, L)]
            for d in range(EMB_DIM):
                ut = plsc.load_gather(ug_v, [jvec, uo + d])
                it = plsc.load_gather(ig_v, [jvec, io + d])
                acc = acc + ut * it
            logits_v[pl.ds(c * CHUNK + g, L)] = acc

    pltpu.sync_copy(logits_v, logits_hbm.at[pl.ds(base, ROWS_PER_WORKER)])


def _sc_logits(ue2, ie2, item_bias, user, item):
    mesh = plsc.VectorSubcoreMesh(core_axis_name="c", subcore_axis_name="s")
    k = pl.kernel(
        _sc_logits_kernel,
        out_type=jax.ShapeDtypeStruct((BATCH,), jnp.float32),
        mesh=mesh,
        compiler_params=pltpu.CompilerParams(needs_layout_passes=False),
        scratch_types=[
            pltpu.VMEM((ROWS_PER_WORKER,), jnp.int32),
            pltpu.VMEM((ROWS_PER_WORKER,), jnp.int32),
            pltpu.VMEM((ROWS_PER_WORKER,), jnp.int32),
            pltpu.VMEM((ROWS_PER_WORKER,), jnp.int32),
            pltpu.VMEM((ROWS_PER_WORKER,), jnp.int32),
            pltpu.VMEM((ROWS_PER_WORKER,), jnp.int32),
            pltpu.VMEM((ROWS_PER_WORKER,), jnp.float32),
            pltpu.VMEM((ROWS_PER_WORKER,), jnp.float32),
            pltpu.VMEM((CHUNK, PHYS_W), jnp.float32),
            pltpu.VMEM((CHUNK, PHYS_W), jnp.float32),
            pltpu.SemaphoreType.DMA,
            pltpu.SemaphoreType.DMA,
            pltpu.SemaphoreType.DMA,
        ],
    )
    return k(ue2, ie2, item_bias, user, item)


def _tc_relayout_kernel(xu_ref, xi_ref, ou_ref, oi_ref):
    for x_ref, o_ref in ((xu_ref, ou_ref), (xi_ref, oi_ref)):
        x = x_ref[...]
        y = jnp.concatenate(
            [x[:, k * PANEL:(k + 1) * PANEL] for k in range(8)],
            axis=0)
        o_ref[...] = y.T


def _tc_relayout(ue_t, ie_t):
    return pl.pallas_call(
        _tc_relayout_kernel,
        grid=(N_BLK,),
        in_specs=[pl.BlockSpec((EMB_DIM, RELAYOUT_W), lambda c: (0, c)),
                  pl.BlockSpec((EMB_DIM, RELAYOUT_W), lambda c: (0, c))],
        out_specs=[pl.BlockSpec((PANEL, PHYS_W), lambda c: (c, 0)),
                   pl.BlockSpec((PANEL, PHYS_W), lambda c: (c, 0))],
        out_shape=[jax.ShapeDtypeStruct((PACK_ROWS, PHYS_W), jnp.float32),
                   jax.ShapeDtypeStruct((PACK_ROWS, PHYS_W), jnp.float32)],
    )(ue_t, ie_t)


def _tc_bce_kernel(x_ref, t_ref, o_ref):
    logits = x_ref[...]
    t = t_ref[...]
    per = (jnp.maximum(logits, 0.0) - logits * t
           + jnp.log1p(jnp.exp(-jnp.abs(logits))))
    o_ref[...] = jnp.reshape(jnp.sum(per) * (1.0 / BATCH), (1, 1))


@jax.jit
def kernel(user, item, label, user_embeddings, item_embeddings, item_bias):
    user = user.astype(jnp.int32)
    item = item.astype(jnp.int32)
    ue2, ie2 = _tc_relayout(user_embeddings.T, item_embeddings.T)
    logits = _sc_logits(ue2, ie2, item_bias, user, item)
    loss = pl.pallas_call(
        _tc_bce_kernel,
        out_shape=jax.ShapeDtypeStruct((1, 1), jnp.float32),
    )(logits.reshape(128, 128), label.reshape(128, 128))
    return loss.reshape(())
